# Initial kernel scaffold; baseline (speedup 1.0000x reference)
import jax
import jax.numpy as jnp
from jax.experimental import pallas as pl


def kernel(x, edge_index, W0_0, b0_0, W0_1, b0_1, W0_2, b0_2, bn_gamma, bn_beta, bn_rm, bn_rv, W1_0, b1_0, W1_1, b1_1, W1_2, b1_2):
    raise NotImplementedError("write your pallas kernel here")


# speedup vs baseline: 23.2899x; 23.2899x over previous
import functools

import jax
import jax.numpy as jnp
from jax import lax
from jax.experimental import pallas as pl
from jax.experimental.pallas import tpu as pltpu
from jax.experimental.pallas import tpu_sc as plsc

N = 10000
E = 320000
NC = 2
NS = 16
NW = NC * NS
CHW = 250
RPT = 624
LO_BASE = NS * RPT
LO = N - LO_BASE
BN_EPS = 1e-5

_sds = jax.ShapeDtypeStruct


def _mesh():
    return plsc.VectorSubcoreMesh(core_axis_name="c", subcore_axis_name="s",
                                  num_cores=NC, num_subcores=NS)


def _make_prop(C, K, TR):
    @functools.partial(
        pl.kernel,
        out_type=_sds((NC, N, C), jnp.float32),
        mesh=_mesh(),
        scratch_types=[
            pltpu.VMEM((CHW, K), jnp.int32),
            pltpu.VMEM((CHW, K), jnp.int32),
            pltpu.VMEM((2, K, C), jnp.float32),
            pltpu.VMEM_SHARED((N, C), jnp.float32),
            pltpu.SemaphoreType.DMA,
            pltpu.SemaphoreType.DMA,
        ],
        compiler_params=pltpu.CompilerParams(use_tc_tiling_on_sc=False),
    )
    def prop(z_hbm, zeros_hbm, row_hbm, col_hbm, out_hbm,
             rowv, colv, gbuf, acc, sem0, sem1):
        cid = lax.axis_index("c")
        sid = lax.axis_index("s")
        wid = cid * NS + sid
        base = sid * RPT
        pltpu.sync_copy(zeros_hbm.at[pl.ds(base, RPT)], acc.at[pl.ds(base, RPT)])

        @pl.when(sid == 0)
        def _():
            pltpu.sync_copy(zeros_hbm.at[pl.ds(LO_BASE, LO)],
                            acc.at[pl.ds(LO_BASE, LO)])

        pltpu.sync_copy(row_hbm.at[wid], rowv)
        pltpu.sync_copy(col_hbm.at[wid], colv)
        plsc.subcore_barrier()

        pltpu.async_copy(z_hbm.at[rowv.at[0]], gbuf.at[0], sem0)

        def step(i, carry):
            jj = 2 * i
            pltpu.async_copy(z_hbm.at[rowv.at[jj + 1]], gbuf.at[1], sem1)
            pltpu.make_async_copy(z_hbm.at[rowv.at[jj]], gbuf.at[0], sem0).wait()
            pltpu.sync_copy(gbuf.at[0], acc.at[colv.at[jj]], add=True)
            pltpu.async_copy(z_hbm.at[rowv.at[jj + 2]], gbuf.at[0], sem0)
            pltpu.make_async_copy(z_hbm.at[rowv.at[jj + 1]], gbuf.at[1], sem1).wait()
            pltpu.sync_copy(gbuf.at[1], acc.at[colv.at[jj + 1]], add=True)
            return carry

        lax.fori_loop(0, CHW // 2 - 1, step, 0)
        pltpu.async_copy(z_hbm.at[rowv.at[CHW - 1]], gbuf.at[1], sem1)
        pltpu.make_async_copy(z_hbm.at[rowv.at[CHW - 2]], gbuf.at[0], sem0).wait()
        pltpu.sync_copy(gbuf.at[0], acc.at[colv.at[CHW - 2]], add=True)
        pltpu.make_async_copy(z_hbm.at[rowv.at[CHW - 1]], gbuf.at[1], sem1).wait()
        pltpu.sync_copy(gbuf.at[1], acc.at[colv.at[CHW - 1]], add=True)

        plsc.subcore_barrier()
        pltpu.sync_copy(acc.at[pl.ds(base, RPT)], out_hbm.at[cid, pl.ds(base, RPT)])

        @pl.when(sid == 0)
        def _():
            pltpu.sync_copy(acc.at[pl.ds(LO_BASE, LO)],
                            out_hbm.at[cid, pl.ds(LO_BASE, LO)])

    return prop


_prop64x2 = _make_prop(64, 80, 2 * N)
_prop64 = _make_prop(64, 40, N)


@functools.partial(
    pl.kernel,
    out_type=_sds((NC, N, 16), jnp.float32),
    mesh=_mesh(),
    scratch_types=[
        pltpu.VMEM((CHW, 40), jnp.int32),
        pltpu.VMEM((40, 16), jnp.float32),
        pltpu.VMEM_SHARED((N, 16), jnp.float32),
    ],
    compiler_params=pltpu.CompilerParams(use_tc_tiling_on_sc=False),
)
def _deg_kernel(ones_hbm, zeros_hbm, col_hbm, out_hbm, colv, obuf, acc):
    cid = lax.axis_index("c")
    sid = lax.axis_index("s")
    wid = cid * NS + sid
    base = sid * RPT
    pltpu.sync_copy(zeros_hbm.at[pl.ds(base, RPT)], acc.at[pl.ds(base, RPT)])

    @pl.when(sid == 0)
    def _():
        pltpu.sync_copy(zeros_hbm.at[pl.ds(LO_BASE, LO)],
                        acc.at[pl.ds(LO_BASE, LO)])

    pltpu.sync_copy(ones_hbm, obuf)
    pltpu.sync_copy(col_hbm.at[wid], colv)
    plsc.subcore_barrier()

    def step(j, carry):
        pltpu.sync_copy(obuf, acc.at[colv.at[j]], add=True)
        return carry

    lax.fori_loop(0, CHW, step, 0)
    plsc.subcore_barrier()
    pltpu.sync_copy(acc.at[pl.ds(base, RPT)], out_hbm.at[cid, pl.ds(base, RPT)])

    @pl.when(sid == 0)
    def _():
        pltpu.sync_copy(acc.at[pl.ds(LO_BASE, LO)],
                        out_hbm.at[cid, pl.ds(LO_BASE, LO)])


R = 2000
G = N // R


def _full(shape):
    return pl.BlockSpec(shape, lambda i: tuple(0 for _ in shape))


def _rows(c):
    return pl.BlockSpec((R, c), lambda i: (i, 0))


def _halves():
    return pl.BlockSpec((2, R, 64), lambda i: (0, i, 0))


def _tc_prep(degp, x, w00, b00):
    def body(degp_r, x_r, w_r, b_r, dis_r, z1s_r, xw0_r):
        deg = degp_r[0, :, 0:1] + degp_r[1, :, 0:1] + 1.0
        dis = lax.rsqrt(deg)
        dis_r[...] = dis
        z1 = x_r[...] * dis
        z1s_r[0] = z1[:, :64]
        z1s_r[1] = z1[:, 64:]
        xw0_r[...] = jnp.dot(x_r[...], w_r[...],
                             preferred_element_type=jnp.float32) + b_r[...]

    return pl.pallas_call(
        body,
        grid=(G,),
        in_specs=[pl.BlockSpec((2, R, 16), lambda i: (0, i, 0)),
                  _rows(128), _full((128, 128)), _full((1, 128))],
        out_specs=[_rows(1), _halves(), _rows(128)],
        out_shape=[_sds((N, 1), jnp.float32),
                   _sds((2, N, 64), jnp.float32),
                   _sds((N, 128), jnp.float32)],
    )(degp, x, w00, b00)


def _tc_combine_a(pa, z1s, dis):
    def body(pa_r, z1s_r, dis_r, p1_r, z2s_r):
        dis_v = dis_r[...]
        acat = jnp.concatenate([pa_r[0] + z1s_r[0], pa_r[1] + z1s_r[1]], axis=1)
        p1 = dis_v * acat
        p1_r[...] = p1
        z2 = dis_v * p1
        z2s_r[0] = z2[:, :64]
        z2s_r[1] = z2[:, 64:]

    return pl.pallas_call(
        body,
        grid=(G,),
        in_specs=[_halves(), _halves(), _rows(1)],
        out_specs=[_rows(128), _halves()],
        out_shape=[_sds((N, 128), jnp.float32), _sds((2, N, 64), jnp.float32)],
    )(pa, z1s, dis)


def _tc_big(pb, z2s, dis, p1, xw0, w01, b01, w02, b02,
            bn_g, bn_b, bn_rm, bn_rv, w10, b10, w12):
    def body(pb_r, z2s_r, dis_r, p1_r, xw0_r, w01_r, b01_r, w02_r, b02_r,
             g_r, be_r, rm_r, rv_r, w10_r, b10_r, w12_r, hw0_r, z3s_r):
        dis_v = dis_r[...]
        p2 = dis_v * jnp.concatenate([pb_r[0] + z2s_r[0], pb_r[1] + z2s_r[1]],
                                     axis=1)
        o2 = jnp.dot(p1_r[...], w01_r[...],
                     preferred_element_type=jnp.float32) + b01_r[...]
        o3 = jnp.dot(p2, w02_r[...],
                     preferred_element_type=jnp.float32) + b02_r[...]
        hcat = jnp.concatenate([xw0_r[...], o2, o3], axis=1)
        scale = g_r[...] * lax.rsqrt(rv_r[...] + BN_EPS)
        shift = be_r[...] - rm_r[...] * scale
        h = jnp.maximum(hcat * scale + shift, 0.0)
        hw0_r[...] = jnp.dot(h, w10_r[...],
                             preferred_element_type=jnp.float32) + b10_r[...]
        z3 = dis_v * jnp.dot(h, w12_r[...], preferred_element_type=jnp.float32)
        z3s_r[0] = z3[:, :64]
        z3s_r[1] = z3[:, 64:]

    return pl.pallas_call(
        body,
        grid=(G,),
        in_specs=[_halves(), _halves(), _rows(1), _rows(128), _rows(128),
                  _full((128, 128)), _full((1, 128)),
                  _full((128, 128)), _full((1, 128)),
                  _full((1, 384)), _full((1, 384)), _full((1, 384)), _full((1, 384)),
                  _full((384, 64)), _full((1, 64)), _full((384, 128))],
        out_specs=[_rows(64), _halves()],
        out_shape=[_sds((N, 64), jnp.float32), _sds((2, N, 64), jnp.float32)],
    )(pb, z2s, dis, p1, xw0, w01, b01, w02, b02,
      bn_g, bn_b, bn_rm, bn_rv, w10, b10, w12)


def _tc_mid(pc, z3s, dis, b11):
    def body(pc_r, z3s_r, dis_r, b11_r, v1b_r, z4_r):
        dis_v = dis_r[...]
        v1b_r[...] = dis_v * (pc_r[0] + z3s_r[0]) + b11_r[...]
        z4_r[...] = dis_v * (dis_v * (pc_r[1] + z3s_r[1]))

    return pl.pallas_call(
        body,
        grid=(G,),
        in_specs=[_halves(), _halves(), _rows(1), _full((1, 64))],
        out_specs=[_rows(64), _rows(64)],
        out_shape=[_sds((N, 64), jnp.float32), _sds((N, 64), jnp.float32)],
    )(pc, z3s, dis, b11)


def _tc_final(pd, z4, dis, hw0, v1b, b12):
    def body(pd_r, z4_r, dis_r, hw0_r, v1b_r, b12_r, out_r):
        w2 = dis_r[...] * (pd_r[0] + pd_r[1] + z4_r[...]) + b12_r[...]
        g = jnp.concatenate([hw0_r[...], v1b_r[...], w2], axis=1)
        m = jnp.max(g, axis=1, keepdims=True)
        e = jnp.exp(g - m)
        s = jnp.sum(e, axis=1, keepdims=True)
        out_r[...] = g - m - jnp.log(s)

    return pl.pallas_call(
        body,
        grid=(G,),
        in_specs=[_halves(), _rows(64), _rows(1), _rows(64), _rows(64),
                  _full((1, 64))],
        out_specs=_rows(192),
        out_shape=_sds((N, 192), jnp.float32),
    )(pd, z4, dis, hw0, v1b, b12)


def kernel(x, edge_index, W0_0, b0_0, W0_1, b0_1, W0_2, b0_2,
           bn_gamma, bn_beta, bn_rm, bn_rv,
           W1_0, b1_0, W1_1, b1_1, W1_2, b1_2):
    row = edge_index[0]
    col = edge_index[1]
    r16 = row.reshape(NS, CHW, 80)
    c16 = col.reshape(NS, CHW, 80)
    row_x2 = jnp.concatenate([r16, r16 + N], axis=0)
    col_x2 = jnp.concatenate([c16, c16], axis=0)
    row_es = row.reshape(NW, CHW, 40)
    col_es = col.reshape(NW, CHW, 40)

    ones16 = jnp.ones((40, 16), jnp.float32)
    zeros16 = jnp.zeros((N, 16), jnp.float32)
    zeros64 = jnp.zeros((N, 64), jnp.float32)

    b1_1r = b1_1.reshape(1, 64)
    b1_2r = b1_2.reshape(1, 64)
    w12 = jnp.concatenate([W1_1, W1_2], axis=1)
    bn4 = [p.reshape(1, 384) for p in (bn_gamma, bn_beta, bn_rm, bn_rv)]

    degp = _deg_kernel(ones16, zeros16, col_es)
    dis, z1s, xw0 = _tc_prep(degp, x, W0_0, b0_0.reshape(1, 128))

    pa = _prop64x2(z1s.reshape(2 * N, 64), zeros64, row_x2, col_x2)
    p1, z2s = _tc_combine_a(pa, z1s, dis)

    pb = _prop64x2(z2s.reshape(2 * N, 64), zeros64, row_x2, col_x2)
    hw0, z3s = _tc_big(pb, z2s, dis, p1, xw0, W0_1, b0_1.reshape(1, 128),
                       W0_2, b0_2.reshape(1, 128), *bn4,
                       W1_0, b1_0.reshape(1, 64), w12)

    pc = _prop64x2(z3s.reshape(2 * N, 64), zeros64, row_x2, col_x2)
    v1b, z4 = _tc_mid(pc, z3s, dis, b1_1r)

    pd = _prop64(z4, zeros64, row_es, col_es)
    return _tc_final(pd, z4, dis, hw0, v1b, b1_2r)
